# baseline (speedup 1.0000x reference)
import functools

import jax
import jax.numpy as jnp
from jax import lax
from jax.experimental import pallas as pl
from jax.experimental.pallas import tpu as pltpu
from jax.experimental.pallas import tpu_sc as plsc

NC = 2
NS = 16
NW = NC * NS
L = 16
DH = 8
DW = 128
DT = 16
SP = DW + 1


def _make_gather(n1, n2, V, D):
    n_dh = D // DH
    n_ih = n1 // DW
    n_dhp = D // DT
    sslabs = n2 * n_dhp
    ss_pw = sslabs // NW
    tstride = D + 1
    n_q = 8
    n_chunk = n_ih // n_q
    mesh = plsc.VectorSubcoreMesh(
        core_axis_name="c", subcore_axis_name="s", num_cores=NC, num_subcores=NS
    )

    @functools.partial(
        pl.kernel,
        out_type=jax.ShapeDtypeStruct((n2, n_dh, n_ih, DH, DW), jnp.float32),
        mesh=mesh,
        compiler_params=pltpu.CompilerParams(
            use_tc_tiling_on_sc=False, needs_layout_passes=False
        ),
        scratch_types=[
            pltpu.VMEM((V * (D + 1),), jnp.float32),
            pltpu.VMEM((2, n1), jnp.int32),
            pltpu.VMEM((2, n_q, DT, SP), jnp.float32),
            pltpu.SemaphoreType.DMA,
            pltpu.SemaphoreType.DMA,
            pltpu.SemaphoreType.DMA,
            pltpu.SemaphoreType.DMA,
        ],
    )
    def gather_kernel(idxt_hbm, table_hbm, out_hbm, tflat_v, icol_v, stage_v,
                      i0, i1, w0, w1):
        isem = [i0, i1]
        wsem = [w0, w1]
        wid = lax.axis_index("s") * NC + lax.axis_index("c")
        base = wid * ss_pw
        pltpu.sync_copy(table_hbm, tflat_v)
        dt_iota = lax.iota(jnp.int32, L)

        def fire_idx(k, b):
            j = (base + k) // n_dhp
            pltpu.async_copy(idxt_hbm.at[j], icol_v.at[b], isem[b])

        def wait_idx(b):
            pltpu.make_async_copy(idxt_hbm.at[0], icol_v.at[b], isem[b]).wait()

        def wait_write(b):
            for _ in range(2):
                pltpu.make_async_copy(
                    stage_v.at[b, :, pl.ds(0, DH), pl.ds(0, DW)],
                    out_hbm.at[0, 0, pl.ds(0, n_q)],
                    wsem[b],
                ).wait()

        def do_superslab(k, ib):
            s = base + k
            j = s // n_dhp
            dhp = s % n_dhp
            wait_idx(ib)

            @pl.when(k + 1 < ss_pw)
            def _():
                fire_idx(k + 1, ib ^ 1)

            for ihq in range(n_chunk):
                sb = ihq & 1
                if ihq >= 2:
                    wait_write(sb)
                else:
                    @pl.when(k > 0)
                    def _():
                        wait_write(sb)

                @plsc.parallel_loop(0, n_q * (DW // L), unroll=4)
                def qg_body(qg):
                    q = qg // (DW // L)
                    g = qg % (DW // L)
                    ih = ihq * n_q + q
                    iv = icol_v[ib, pl.ds(ih * DW + g * L, L)]
                    ivb = iv * tstride + (dhp * DT)
                    qsplat = jnp.full((L,), q, jnp.int32)
                    for kk in range(L):
                        gaddr = jnp.full((L,), ivb[kk], jnp.int32) + dt_iota
                        v = plsc.load_gather(tflat_v, [gaddr])
                        ilsplat = jnp.full((L,), g * L + kk, jnp.int32)
                        plsc.store_scatter(
                            stage_v.at[sb], [qsplat, dt_iota, ilsplat], v
                        )

                for dr in range(DT // DH):
                    dh = dhp * (DT // DH) + dr
                    pltpu.async_copy(
                        stage_v.at[sb, :, pl.ds(dr * DH, DH), pl.ds(0, DW)],
                        out_hbm.at[j, dh, pl.ds(ihq * n_q, n_q)],
                        wsem[sb],
                    )

        fire_idx(0, 0)

        def body(t, carry):
            do_superslab(t * 2, 0)
            do_superslab(t * 2 + 1, 1)
            return carry

        lax.fori_loop(0, ss_pw // 2, body, 0)
        if ss_pw % 2:
            do_superslab(ss_pw - 1, 0)
        wait_write(0)
        wait_write(1)

    return gather_kernel


def kernel(indices, emb_dim, table):
    n1, n2 = indices.shape
    V, D = table.shape
    assert n1 % DW == 0 and D % DT == 0
    assert (n2 * (D // DT)) % NW == 0

    gate = jnp.where(
        jnp.asarray(emb_dim) == D, jnp.float32(1.0), jnp.float32(jnp.nan)
    ).astype(table.dtype)
    table_gated = (table * gate).astype(jnp.float32)
    table_flat = jnp.pad(table_gated, ((0, 0), (0, 1))).reshape(-1)

    phys = _make_gather(n1, n2, V, D)(indices.T, table_flat)
    return phys.transpose(2, 4, 0, 1, 3).reshape(n1, n2, D)


# speedup vs baseline: 1.2034x; 1.2034x over previous
import functools

import jax
import jax.numpy as jnp
from jax import lax
from jax.experimental import pallas as pl
from jax.experimental.pallas import tpu as pltpu
from jax.experimental.pallas import tpu_sc as plsc

NC = 2
NS = 16
NW = NC * NS
L = 16
DH = 8
DW = 128
DT = 16
SP = DW + 1


def _make_gather(n1, n2, V, D):
    n_dh = D // DH
    n_ih = n1 // DW
    n_dhp = D // DT
    sslabs = n2 * n_dhp
    ss_pw = sslabs // NW
    tstride = D + 1
    n_q = 8
    n_chunk = n_ih // n_q
    mesh = plsc.VectorSubcoreMesh(
        core_axis_name="c", subcore_axis_name="s", num_cores=NC, num_subcores=NS
    )

    @functools.partial(
        pl.kernel,
        out_type=jax.ShapeDtypeStruct((n2, n_dh, n_ih, DH, DW), jnp.float32),
        mesh=mesh,
        compiler_params=pltpu.CompilerParams(
            use_tc_tiling_on_sc=False, needs_layout_passes=False
        ),
        scratch_types=[
            pltpu.VMEM((V * (D + 1),), jnp.float32),
            pltpu.VMEM((2, n1), jnp.int32),
            pltpu.VMEM((2, n_q, DT, SP), jnp.float32),
            pltpu.SemaphoreType.DMA,
            pltpu.SemaphoreType.DMA,
            pltpu.SemaphoreType.DMA,
            pltpu.SemaphoreType.DMA,
        ],
    )
    def gather_kernel(idxt_hbm, table_hbm, out_hbm, tflat_v, icol_v, stage_v,
                      i0, i1, w0, w1):
        isem = [i0, i1]
        wsem = [w0, w1]
        wid = lax.axis_index("s") * NC + lax.axis_index("c")
        base = wid * ss_pw
        pltpu.sync_copy(table_hbm, tflat_v)
        dt_iota = lax.iota(jnp.int32, L)

        def fire_idx(k, b):
            j = (base + k) // n_dhp
            pltpu.async_copy(idxt_hbm.at[j], icol_v.at[b], isem[b])

        def wait_idx(b):
            pltpu.make_async_copy(idxt_hbm.at[0], icol_v.at[b], isem[b]).wait()

        def wait_write(b):
            for _ in range(2):
                pltpu.make_async_copy(
                    stage_v.at[b, :, pl.ds(0, DH), pl.ds(0, DW)],
                    out_hbm.at[0, 0, pl.ds(0, n_q)],
                    wsem[b],
                ).wait()

        def do_superslab(k, ib):
            s = base + k
            j = s // n_dhp
            dhp = s % n_dhp
            wait_idx(ib)

            @pl.when(k + 1 < ss_pw)
            def _():
                fire_idx(k + 1, ib ^ 1)

            for ihq in range(n_chunk):
                sb = ihq & 1
                if ihq >= 2:
                    wait_write(sb)
                else:
                    @pl.when(k > 0)
                    def _():
                        wait_write(sb)

                @plsc.parallel_loop(0, n_q * (DW // L), unroll=2)
                def qg_body(qg):
                    q = qg // (DW // L)
                    g = qg % (DW // L)
                    ih = ihq * n_q + q
                    iv = icol_v[ib, pl.ds(ih * DW + g * L, L)]
                    ivb = iv * tstride + (dhp * DT)
                    for kk in range(L):
                        gaddr = jnp.full((L,), ivb[kk], jnp.int32) + dt_iota
                        v = plsc.load_gather(tflat_v, [gaddr])
                        ilsplat = jnp.full((L,), g * L + kk, jnp.int32)
                        plsc.store_scatter(
                            stage_v.at[sb].at[q], [dt_iota, ilsplat], v
                        )

                for dr in range(DT // DH):
                    dh = dhp * (DT // DH) + dr
                    pltpu.async_copy(
                        stage_v.at[sb, :, pl.ds(dr * DH, DH), pl.ds(0, DW)],
                        out_hbm.at[j, dh, pl.ds(ihq * n_q, n_q)],
                        wsem[sb],
                    )

        fire_idx(0, 0)

        def body(t, carry):
            do_superslab(t * 2, 0)
            do_superslab(t * 2 + 1, 1)
            return carry

        lax.fori_loop(0, ss_pw // 2, body, 0)
        if ss_pw % 2:
            do_superslab(ss_pw - 1, 0)
        wait_write(0)
        wait_write(1)

    return gather_kernel


def kernel(indices, emb_dim, table):
    n1, n2 = indices.shape
    V, D = table.shape
    assert n1 % DW == 0 and D % DT == 0
    assert (n2 * (D // DT)) % NW == 0

    gate = jnp.where(
        jnp.asarray(emb_dim) == D, jnp.float32(1.0), jnp.float32(jnp.nan)
    ).astype(table.dtype)
    table_gated = (table * gate).astype(jnp.float32)
    table_flat = jnp.pad(table_gated, ((0, 0), (0, 1))).reshape(-1)

    phys = _make_gather(n1, n2, V, D)(indices.T, table_flat)
    return phys.transpose(2, 4, 0, 1, 3).reshape(n1, n2, D)


# speedup vs baseline: 1.3777x; 1.1448x over previous
import functools

import jax
import jax.numpy as jnp
from jax import lax
from jax.experimental import pallas as pl
from jax.experimental.pallas import tpu as pltpu
from jax.experimental.pallas import tpu_sc as plsc

NC = 2
NS = 16
NW = NC * NS
L = 16
DH = 8
DW = 128
DT = 16
SP = DW + 1


def _make_gather(n1, n2, V, D):
    n_dh = D // DH
    n_ih = n1 // DW
    n_dhp = D // DT
    sslabs = n2 * n_dhp
    ss_pw = sslabs // NW
    tstride = D + 1
    n_q = 8
    n_chunk = n_ih // n_q
    mesh = plsc.VectorSubcoreMesh(
        core_axis_name="c", subcore_axis_name="s", num_cores=NC, num_subcores=NS
    )

    @functools.partial(
        pl.kernel,
        out_type=jax.ShapeDtypeStruct((n2, n_dh, n_ih, DH, DW), jnp.float32),
        mesh=mesh,
        compiler_params=pltpu.CompilerParams(
            use_tc_tiling_on_sc=False, needs_layout_passes=False
        ),
        scratch_types=[
            pltpu.VMEM((V * (D + 1),), jnp.float32),
            pltpu.VMEM((2, n1), jnp.int32),
            pltpu.VMEM((2, n_q, DT, SP), jnp.float32),
            pltpu.SemaphoreType.DMA,
            pltpu.SemaphoreType.DMA,
            pltpu.SemaphoreType.DMA,
            pltpu.SemaphoreType.DMA,
        ],
    )
    def gather_kernel(idxt_hbm, table_hbm, out_hbm, tflat_v, icol_v, stage_v,
                      i0, i1, w0, w1):
        isem = [i0, i1]
        wsem = [w0, w1]
        wid = lax.axis_index("s") * NC + lax.axis_index("c")
        base = wid * ss_pw
        pltpu.sync_copy(table_hbm, tflat_v)
        dt_iota = lax.iota(jnp.int32, L)

        def fire_idx(k, b):
            j = (base + k) // n_dhp
            pltpu.async_copy(idxt_hbm.at[j], icol_v.at[b], isem[b])

        def wait_idx(b):
            pltpu.make_async_copy(idxt_hbm.at[0], icol_v.at[b], isem[b]).wait()

        def wait_write(b):
            for _ in range(2):
                pltpu.make_async_copy(
                    stage_v.at[b, :, pl.ds(0, DH), pl.ds(0, DW)],
                    out_hbm.at[0, 0, pl.ds(0, n_q)],
                    wsem[b],
                ).wait()

        def do_superslab(k, ib):
            s = base + k
            j = s // n_dhp
            dhp = s % n_dhp
            wait_idx(ib)

            @pl.when(k + 1 < ss_pw)
            def _():
                fire_idx(k + 1, ib ^ 1)

            for ihq in range(n_chunk):
                sb = ihq & 1
                if ihq >= 2:
                    wait_write(sb)
                else:
                    @pl.when(k > 0)
                    def _():
                        wait_write(sb)

                @plsc.parallel_loop(0, n_q * (DW // L), unroll=2)
                def qg_body(qg):
                    q = qg // (DW // L)
                    g = qg % (DW // L)
                    ih = ihq * n_q + q
                    iv = icol_v[ib, pl.ds(ih * DW + g * L, L)]
                    ivb = iv * tstride + (dhp * DT)
                    qsplat = jnp.full((L,), q, jnp.int32)
                    for kk in range(L):
                        gaddr = jnp.full((L,), ivb[kk], jnp.int32) + dt_iota
                        v = plsc.load_gather(tflat_v, [gaddr])
                        ilsplat = jnp.full((L,), g * L + kk, jnp.int32)
                        plsc.store_scatter(
                            stage_v.at[sb], [qsplat, dt_iota, ilsplat], v
                        )

                for dr in range(DT // DH):
                    dh = dhp * (DT // DH) + dr
                    pltpu.async_copy(
                        stage_v.at[sb, :, pl.ds(dr * DH, DH), pl.ds(0, DW)],
                        out_hbm.at[j, dh, pl.ds(ihq * n_q, n_q)],
                        wsem[sb],
                    )

        fire_idx(0, 0)

        def body(t, carry):
            do_superslab(t * 2, 0)
            do_superslab(t * 2 + 1, 1)
            return carry

        lax.fori_loop(0, ss_pw // 2, body, 0)
        if ss_pw % 2:
            do_superslab(ss_pw - 1, 0)
        wait_write(0)
        wait_write(1)

    return gather_kernel


def kernel(indices, emb_dim, table):
    n1, n2 = indices.shape
    V, D = table.shape
    assert n1 % DW == 0 and D % DT == 0
    assert (n2 * (D // DT)) % NW == 0

    gate = jnp.where(
        jnp.asarray(emb_dim) == D, jnp.float32(1.0), jnp.float32(jnp.nan)
    ).astype(table.dtype)
    table_gated = (table * gate).astype(jnp.float32)
    table_flat = jnp.pad(table_gated, ((0, 0), (0, 1))).reshape(-1)

    phys = _make_gather(n1, n2, V, D)(indices.T, table_flat)
    return phys.transpose(2, 4, 0, 1, 3).reshape(n1, n2, D)


# speedup vs baseline: 1.4008x; 1.0168x over previous
import functools

import jax
import jax.numpy as jnp
from jax import lax
from jax.experimental import pallas as pl
from jax.experimental.pallas import tpu as pltpu
from jax.experimental.pallas import tpu_sc as plsc

NC = 2
NS = 16
NW = NC * NS
L = 16
DH = 8
DW = 128
DT = 16
SP = DW + 1


def _make_gather(n1, n2, V, D):
    n_dh = D // DH
    n_ih = n1 // DW
    n_dhp = D // DT
    sslabs = n2 * n_dhp
    ss_pw = sslabs // NW
    tstride = D + 1
    n_q = 16
    n_chunk = n_ih // n_q
    mesh = plsc.VectorSubcoreMesh(
        core_axis_name="c", subcore_axis_name="s", num_cores=NC, num_subcores=NS
    )

    @functools.partial(
        pl.kernel,
        out_type=jax.ShapeDtypeStruct((n2, n_dh, n_ih, DH, DW), jnp.float32),
        mesh=mesh,
        compiler_params=pltpu.CompilerParams(
            use_tc_tiling_on_sc=False, needs_layout_passes=False
        ),
        scratch_types=[
            pltpu.VMEM((V * (D + 1),), jnp.float32),
            pltpu.VMEM((2, n1), jnp.int32),
            pltpu.VMEM((2, n_q, DT, SP), jnp.float32),
            pltpu.SemaphoreType.DMA,
            pltpu.SemaphoreType.DMA,
            pltpu.SemaphoreType.DMA,
            pltpu.SemaphoreType.DMA,
        ],
    )
    def gather_kernel(idxt_hbm, table_hbm, out_hbm, tflat_v, icol_v, stage_v,
                      i0, i1, w0, w1):
        isem = [i0, i1]
        wsem = [w0, w1]
        wid = lax.axis_index("s") * NC + lax.axis_index("c")
        base = wid * ss_pw
        pltpu.sync_copy(table_hbm, tflat_v)
        dt_iota = lax.iota(jnp.int32, L)

        def fire_idx(k, b):
            j = (base + k) // n_dhp
            pltpu.async_copy(idxt_hbm.at[j], icol_v.at[b], isem[b])

        def wait_idx(b):
            pltpu.make_async_copy(idxt_hbm.at[0], icol_v.at[b], isem[b]).wait()

        def wait_write(b):
            for _ in range(2):
                pltpu.make_async_copy(
                    stage_v.at[b, :, pl.ds(0, DH), pl.ds(0, DW)],
                    out_hbm.at[0, 0, pl.ds(0, n_q)],
                    wsem[b],
                ).wait()

        def do_superslab(k, ib):
            s = base + k
            j = s // n_dhp
            dhp = s % n_dhp
            wait_idx(ib)

            @pl.when(k + 1 < ss_pw)
            def _():
                fire_idx(k + 1, ib ^ 1)

            for ihq in range(n_chunk):
                sb = ihq & 1
                if ihq >= 2:
                    wait_write(sb)
                else:
                    @pl.when(k > 0)
                    def _():
                        wait_write(sb)

                @plsc.parallel_loop(0, n_q * (DW // L), unroll=2)
                def qg_body(qg):
                    q = qg // (DW // L)
                    g = qg % (DW // L)
                    ih = ihq * n_q + q
                    iv = icol_v[ib, pl.ds(ih * DW + g * L, L)]
                    ivb = iv * tstride + (dhp * DT)
                    qsplat = jnp.full((L,), q, jnp.int32)
                    for kk in range(L):
                        gaddr = jnp.full((L,), ivb[kk], jnp.int32) + dt_iota
                        v = plsc.load_gather(tflat_v, [gaddr])
                        ilsplat = jnp.full((L,), g * L + kk, jnp.int32)
                        plsc.store_scatter(
                            stage_v.at[sb], [qsplat, dt_iota, ilsplat], v
                        )

                for dr in range(DT // DH):
                    dh = dhp * (DT // DH) + dr
                    pltpu.async_copy(
                        stage_v.at[sb, :, pl.ds(dr * DH, DH), pl.ds(0, DW)],
                        out_hbm.at[j, dh, pl.ds(ihq * n_q, n_q)],
                        wsem[sb],
                    )

        fire_idx(0, 0)

        def body(t, carry):
            do_superslab(t * 2, 0)
            do_superslab(t * 2 + 1, 1)
            return carry

        lax.fori_loop(0, ss_pw // 2, body, 0)
        if ss_pw % 2:
            do_superslab(ss_pw - 1, 0)
        wait_write(0)
        wait_write(1)

    return gather_kernel


def kernel(indices, emb_dim, table):
    n1, n2 = indices.shape
    V, D = table.shape
    assert n1 % DW == 0 and D % DT == 0
    assert (n2 * (D // DT)) % NW == 0

    gate = jnp.where(
        jnp.asarray(emb_dim) == D, jnp.float32(1.0), jnp.float32(jnp.nan)
    ).astype(table.dtype)
    table_gated = (table * gate).astype(jnp.float32)
    table_flat = jnp.pad(table_gated, ((0, 0), (0, 1))).reshape(-1)

    phys = _make_gather(n1, n2, V, D)(indices.T, table_flat)
    return phys.transpose(2, 4, 0, 1, 3).reshape(n1, n2, D)
